# baseline (speedup 1.0000x reference)
import functools

import jax
import jax.numpy as jnp
from jax import lax
from jax.experimental import pallas as pl
from jax.experimental.pallas import tpu as pltpu
from jax.experimental.pallas import tpu_sc as plsc

NC = 2
NS = 16
NW = NC * NS

IDXW = 128


@functools.lru_cache(maxsize=None)
def _make_gather(B, V, D, chunk):
    assert B % (NW * chunk) == 0 and chunk % IDXW == 0
    rows_per_w = B // NW
    steps = rows_per_w // chunk
    gpc = chunk // IDXW

    mesh = plsc.VectorSubcoreMesh(
        core_axis_name="c", subcore_axis_name="s",
        num_cores=NC, num_subcores=NS)

    @functools.partial(
        pl.kernel,
        mesh=mesh,
        out_type=jax.ShapeDtypeStruct((B, D), jnp.float32),
        scratch_types=[
            pltpu.VMEM((gpc, IDXW), jnp.int32),
            pltpu.VMEM((chunk, D), jnp.float32),
            pltpu.SemaphoreType.DMA,
        ],
        compiler_params=pltpu.CompilerParams(use_tc_tiling_on_sc=False),
    )
    def gather_kernel(table_hbm, idx_hbm, out_hbm, idx_v, rows_v, sem):
        wid = lax.axis_index("s") * NC + lax.axis_index("c")
        irow0 = wid * (rows_per_w // IDXW)
        base = wid * rows_per_w

        def step(s, _):
            pltpu.sync_copy(idx_hbm.at[pl.ds(irow0 + s * gpc, gpc)], idx_v)
            copies = [
                pltpu.async_copy(
                    table_hbm.at[idx_v.at[j]],
                    rows_v.at[pl.ds(j * IDXW, IDXW)],
                    sem,
                )
                for j in range(gpc)
            ]
            for c in copies:
                c.wait()
            pltpu.sync_copy(rows_v, out_hbm.at[pl.ds(base + s * chunk, chunk)])
            return 0

        lax.fori_loop(0, steps, step, 0)

    return gather_kernel


def kernel(token_ids, weight):
    B = token_ids.size
    V, D = weight.shape
    idx = token_ids.reshape(B // IDXW, IDXW).astype(jnp.int32)
    out = _make_gather(B, V, D, 1024)(weight, idx)
    return out.reshape(*token_ids.shape, D)


# speedup vs baseline: 1.0303x; 1.0303x over previous
import functools

import jax
import jax.numpy as jnp
from jax import lax
from jax.experimental import pallas as pl
from jax.experimental.pallas import tpu as pltpu
from jax.experimental.pallas import tpu_sc as plsc

NC = 2
NS = 16
NW = NC * NS

IDXW = 128


@functools.lru_cache(maxsize=None)
def _make_gather(B, V, D, chunk):
    assert B % (NW * chunk) == 0 and chunk % IDXW == 0
    rows_per_w = B // NW
    steps = rows_per_w // chunk
    assert steps % 2 == 0
    gpc = chunk // IDXW

    mesh = plsc.VectorSubcoreMesh(
        core_axis_name="c", subcore_axis_name="s",
        num_cores=NC, num_subcores=NS)

    @functools.partial(
        pl.kernel,
        mesh=mesh,
        out_type=jax.ShapeDtypeStruct((B, D), jnp.float32),
        scratch_types=[
            pltpu.VMEM((2, chunk), jnp.int32),
            pltpu.VMEM((2, chunk, D), jnp.float32),
            pltpu.SemaphoreType.DMA,
            pltpu.SemaphoreType.DMA,
            pltpu.SemaphoreType.DMA,
            pltpu.SemaphoreType.DMA,
            pltpu.SemaphoreType.DMA,
            pltpu.SemaphoreType.DMA,
        ],
        compiler_params=pltpu.CompilerParams(use_tc_tiling_on_sc=False),
    )
    def gather_kernel(table_hbm, idx_hbm, out_hbm, idx_v, rows_v,
                      isem0, isem1, gsem0, gsem1, osem0, osem1):
        isem = (isem0, isem1)
        gsem = (gsem0, gsem1)
        osem = (osem0, osem1)
        wid = lax.axis_index("s") * NC + lax.axis_index("c")
        base = wid * rows_per_w

        def idx_fetch(s, p):
            pltpu.async_copy(
                idx_hbm.at[pl.ds(base + s * chunk, chunk)], idx_v.at[p],
                isem[p])

        def idx_wait(p):
            pltpu.make_async_copy(
                idx_hbm.at[pl.ds(0, chunk)], idx_v.at[p], isem[p]).wait()

        def out_wait(p):
            pltpu.make_async_copy(
                rows_v.at[p], out_hbm.at[pl.ds(0, chunk)], osem[p]).wait()

        def do_step(s, p):
            idx_wait(p)
            @pl.when(s >= 2)
            def _():
                out_wait(p)
            copies = [
                pltpu.async_copy(
                    table_hbm.at[idx_v.at[p, pl.ds(j * IDXW, IDXW)]],
                    rows_v.at[p, pl.ds(j * IDXW, IDXW)],
                    gsem[p],
                )
                for j in range(gpc)
            ]
            for c in copies:
                c.wait()

            @pl.when(s + 2 < steps)
            def _():
                idx_fetch(s + 2, p)

            pltpu.async_copy(
                rows_v.at[p], out_hbm.at[pl.ds(base + s * chunk, chunk)],
                osem[p])

        idx_fetch(0, 0)
        idx_fetch(1, 1)

        def body(g, _):
            do_step(2 * g, 0)
            do_step(2 * g + 1, 1)
            return 0

        lax.fori_loop(0, steps // 2, body, 0)
        out_wait(0)
        out_wait(1)

    return gather_kernel


def kernel(token_ids, weight):
    B = token_ids.size
    V, D = weight.shape
    idx = token_ids.reshape(B).astype(jnp.int32)
    out = _make_gather(B, V, D, 640)(weight, idx)
    return out.reshape(*token_ids.shape, D)


# speedup vs baseline: 1.6932x; 1.6434x over previous
import functools

import jax
import jax.numpy as jnp
from jax import lax
from jax.experimental import pallas as pl
from jax.experimental.pallas import tpu as pltpu
from jax.experimental.pallas import tpu_sc as plsc

NC = 2
NS = 16
NW = NC * NS

IDXW = 128


@functools.lru_cache(maxsize=None)
def _make_gather(B, V, D, chunk):
    assert B % (NW * chunk) == 0 and chunk % IDXW == 0
    rows_per_w = B // NW
    steps = rows_per_w // chunk
    assert steps % 2 == 0
    gpc = chunk // IDXW

    mesh = plsc.VectorSubcoreMesh(
        core_axis_name="c", subcore_axis_name="s",
        num_cores=NC, num_subcores=NS)

    @functools.partial(
        pl.kernel,
        mesh=mesh,
        out_type=jax.ShapeDtypeStruct((B, 2 * D), jnp.float32),
        scratch_types=[
            pltpu.VMEM((2, chunk), jnp.int32),
            pltpu.VMEM((2, chunk, D), jnp.float32),
            pltpu.SemaphoreType.DMA,
            pltpu.SemaphoreType.DMA,
            pltpu.SemaphoreType.DMA,
            pltpu.SemaphoreType.DMA,
            pltpu.SemaphoreType.DMA,
            pltpu.SemaphoreType.DMA,
        ],
        compiler_params=pltpu.CompilerParams(use_tc_tiling_on_sc=False),
    )
    def gather_kernel(table_hbm, idx_hbm, out_hbm, idx_v, rows_v,
                      isem0, isem1, gsem0, gsem1, osem0, osem1):
        isem = (isem0, isem1)
        gsem = (gsem0, gsem1)
        osem = (osem0, osem1)
        wid = lax.axis_index("s") * NC + lax.axis_index("c")
        base = wid * rows_per_w

        def idx_fetch(s, p):
            pltpu.async_copy(
                idx_hbm.at[pl.ds(base + s * chunk, chunk)], idx_v.at[p],
                isem[p])

        def idx_wait(p):
            pltpu.make_async_copy(
                idx_hbm.at[pl.ds(0, chunk)], idx_v.at[p], isem[p]).wait()

        def out_wait(p):
            pltpu.make_async_copy(
                rows_v.at[p],
                out_hbm.at[pl.ds(0, chunk), pl.ds(0, D)], osem[p]).wait()

        def do_step(s, p):
            idx_wait(p)
            @pl.when(s >= 2)
            def _():
                out_wait(p)
            copies = [
                pltpu.async_copy(
                    table_hbm.at[idx_v.at[p, pl.ds(j * IDXW, IDXW)]],
                    rows_v.at[p, pl.ds(j * IDXW, IDXW)],
                    gsem[p],
                )
                for j in range(gpc)
            ]
            for c in copies:
                c.wait()

            @pl.when(s + 2 < steps)
            def _():
                idx_fetch(s + 2, p)

            pltpu.async_copy(
                rows_v.at[p],
                out_hbm.at[pl.ds(base + s * chunk, chunk), pl.ds(0, D)],
                osem[p])

        idx_fetch(0, 0)
        idx_fetch(1, 1)

        def body(g, _):
            do_step(2 * g, 0)
            do_step(2 * g + 1, 1)
            return 0

        lax.fori_loop(0, steps // 2, body, 0)
        out_wait(0)
        out_wait(1)

    return gather_kernel


def kernel(token_ids, weight):
    B = token_ids.size
    V, D = weight.shape
    idx = token_ids.reshape(B).astype(jnp.int32)
    out = _make_gather(B, V, D, 640)(weight, idx)
    return out[:, :D].reshape(*token_ids.shape, D)
